# baseline (speedup 1.0000x reference)
import functools

import jax
import jax.numpy as jnp
from jax import lax
from jax.experimental import pallas as pl
from jax.experimental.pallas import tpu as pltpu
from jax.experimental.pallas import tpu_sc as plsc

_NC, _NS, _L = 2, 16, 16
_NW = _NC * _NS

_A1 = 64000
_A2 = 64000
_C1 = 200
_C2 = 80


def _reduce_stream(src, wid, rows_per_w, chunk, bufs, sems, acc, out_row, ngrp):
    n = rows_per_w // chunk
    base = wid * rows_per_w

    def dma(c):
        off = pl.multiple_of(base + c * chunk, 8)
        return pltpu.async_copy(src.at[pl.ds(off, chunk)], bufs[c % 2], sems[c % 2])

    h = dma(0)
    accs = tuple(jnp.zeros((_L,), jnp.float32) for _ in range(ngrp))
    for c in range(n):
        nh = dma(c + 1) if c + 1 < n else None
        h.wait()
        buf = bufs[c % 2]

        def row(r, accs):
            r2 = 2 * r
            accs = tuple(accs[g] + buf[r2, pl.ds(g * _L, _L)] for g in range(ngrp))
            return tuple(accs[g] + buf[r2 + 1, pl.ds(g * _L, _L)] for g in range(ngrp))

        accs = lax.fori_loop(0, chunk // 2, row, accs)
        h = nh
    for g in range(ngrp):
        acc[pl.ds(g * _L, _L)] = accs[g]
    pltpu.sync_copy(acc, out_row)


def _sc_reduce_body(nbr1, nbr2, p1, p2, buf1a, buf1b, buf2a, buf2b, a1, a2,
                    sem0, sem1):
    wid = lax.axis_index("s") * _NC + lax.axis_index("c")
    _reduce_stream(nbr1, wid, _A1 // _NW, _C1, (buf1a, buf1b), (sem0, sem1),
                   a1, p1.at[wid], 8)
    _reduce_stream(nbr2, wid, _A2 // _NW, _C2, (buf2a, buf2b), (sem0, sem1),
                   a2, p2.at[wid], 16)


def _sc_reduce(nbr1, nbr2):
    feat = nbr1.shape[1]
    hid = nbr2.shape[1]
    mesh = plsc.VectorSubcoreMesh(core_axis_name="c", subcore_axis_name="s")
    f = pl.kernel(
        _sc_reduce_body,
        mesh=mesh,
        out_type=[
            jax.ShapeDtypeStruct((_NW, feat), jnp.float32),
            jax.ShapeDtypeStruct((_NW, hid), jnp.float32),
        ],
        scratch_types=[
            pltpu.VMEM((_C1, feat), jnp.float32),
            pltpu.VMEM((_C1, feat), jnp.float32),
            pltpu.VMEM((_C2, hid), jnp.float32),
            pltpu.VMEM((_C2, hid), jnp.float32),
            pltpu.VMEM((feat,), jnp.float32),
            pltpu.VMEM((hid,), jnp.float32),
            pltpu.SemaphoreType.DMA,
            pltpu.SemaphoreType.DMA,
        ],
    )
    return f(nbr1, nbr2)


def _tc_reduce_body(nbr1, nbr2, q1, q2):
    i = pl.program_id(0)

    @pl.when(i == 0)
    def _init():
        q1[...] = jnp.zeros_like(q1)
        q2[...] = jnp.zeros_like(q2)

    q1[...] += jnp.sum(nbr1[...], axis=0, keepdims=True)
    q2[...] += jnp.sum(nbr2[...], axis=0, keepdims=True)


def _tc_reduce(nbr1, nbr2, blk=1600):
    n_nbr, feat = nbr1.shape
    hid = nbr2.shape[1]
    off1 = _A1 // blk
    off2 = _A2 // blk
    steps = (n_nbr - _A1) // blk
    body = _tc_reduce_body
    return pl.pallas_call(
        body,
        grid=(steps,),
        in_specs=[
            pl.BlockSpec((blk, feat), lambda i: (off1 + i, 0)),
            pl.BlockSpec((blk, hid), lambda i: (off2 + i, 0)),
        ],
        out_specs=[
            pl.BlockSpec((1, feat), lambda i: (0, 0)),
            pl.BlockSpec((1, hid), lambda i: (0, 0)),
        ],
        out_shape=[
            jax.ShapeDtypeStruct((1, feat), jnp.float32),
            jax.ShapeDtypeStruct((1, hid), jnp.float32),
        ],
        compiler_params=pltpu.CompilerParams(
            dimension_semantics=("arbitrary",),
        ),
    )(nbr1, nbr2)


def _tc_dense_body(node, p1, p2, q1, q2,
                   w_self1, b_self1, w_nbr1, b_nbr1, g1, be1,
                   w_self2, b_self2, w_nbr2, b_nbr2, g2, be2,
                   out,
                   c1s, c2s, ssq, h2buf,
                   *, D, n_nbr, dense_blk, eps):
    i = pl.program_id(0)

    @pl.when(i == 0)
    def _bias():
        agg1 = (jnp.sum(p1[...], axis=0, keepdims=True) + q1[...]) * (1.0 / n_nbr)
        agg2 = (jnp.sum(p2[...], axis=0, keepdims=True) + q2[...]) * (1.0 / n_nbr)
        c1s[...] = (jnp.dot(agg1, w_nbr1[...], preferred_element_type=jnp.float32)
                    + b_self1[...] + b_nbr1[...])
        c2s[...] = (jnp.dot(agg2, w_nbr2[...], preferred_element_type=jnp.float32)
                    + b_self2[...] + b_nbr2[...])
        ssq[...] = jnp.zeros_like(ssq)

    @pl.when(i < D)
    def _dense():
        j = i
        x = jnp.dot(node[...], w_self1[...], preferred_element_type=jnp.float32)
        x = x + c1s[...]
        mu = jnp.mean(x, axis=-1, keepdims=True)
        var = jnp.mean((x - mu) ** 2, axis=-1, keepdims=True)
        x = (x - mu) * lax.rsqrt(var + eps) * g1[...] + be1[...]
        h1 = jnp.maximum(x, 0.0)
        y = jnp.dot(h1, w_self2[...], preferred_element_type=jnp.float32)
        y = y + c2s[...]
        mu2 = jnp.mean(y, axis=-1, keepdims=True)
        var2 = jnp.mean((y - mu2) ** 2, axis=-1, keepdims=True)
        y = (y - mu2) * lax.rsqrt(var2 + eps) * g2[...] + be2[...]
        h2 = jnp.maximum(y, 0.0)
        h2buf[pl.ds(j * dense_blk, dense_blk), :] = h2
        ssq[...] += jnp.sum(h2 * h2, axis=0, keepdims=True)

    @pl.when(i >= D)
    def _emit():
        j = i - D
        inv = 1.0 / jnp.maximum(jnp.sqrt(ssq[...]), 1e-12)
        out[...] = h2buf[pl.ds(j * dense_blk, dense_blk), :] * inv


def _tc_dense(node_feat, p1, p2, q1, q2,
              W_self1, b_self1, W_nbr1, b_nbr1, g1, be1,
              W_self2, b_self2, W_nbr2, b_nbr2, g2, be2, n_nbr):
    n_nodes, feat = node_feat.shape
    hid = W_self1.shape[1]
    emb = W_self2.shape[1]
    dense_blk = 1000 if n_nodes % 1000 == 0 else n_nodes
    D = n_nodes // dense_blk

    b_self1 = b_self1.reshape(1, hid)
    b_nbr1 = b_nbr1.reshape(1, hid)
    g1 = g1.reshape(1, hid)
    be1 = be1.reshape(1, hid)
    b_self2 = b_self2.reshape(1, emb)
    b_nbr2 = b_nbr2.reshape(1, emb)
    g2 = g2.reshape(1, emb)
    be2 = be2.reshape(1, emb)

    def node_map(i):
        return (jnp.clip(i, 0, D - 1), 0)

    def out_map(i):
        return (jnp.clip(i - D, 0, D - 1), 0)

    full = lambda s: pl.BlockSpec(s, lambda i: (0, 0))

    body = functools.partial(_tc_dense_body, D=D, n_nbr=n_nbr,
                             dense_blk=dense_blk, eps=1e-5)

    return pl.pallas_call(
        body,
        grid=(2 * D,),
        in_specs=[
            pl.BlockSpec((dense_blk, feat), node_map),
            full((_NW, feat)), full((_NW, hid)),
            full((1, feat)), full((1, hid)),
            full((feat, hid)), full((1, hid)), full((feat, hid)), full((1, hid)),
            full((1, hid)), full((1, hid)),
            full((hid, emb)), full((1, emb)), full((hid, emb)), full((1, emb)),
            full((1, emb)), full((1, emb)),
        ],
        out_specs=pl.BlockSpec((dense_blk, emb), out_map),
        out_shape=jax.ShapeDtypeStruct((n_nodes, emb), jnp.float32),
        scratch_shapes=[
            pltpu.VMEM((1, hid), jnp.float32),
            pltpu.VMEM((1, emb), jnp.float32),
            pltpu.VMEM((1, emb), jnp.float32),
            pltpu.VMEM((n_nodes, emb), jnp.float32),
        ],
        compiler_params=pltpu.CompilerParams(
            dimension_semantics=("arbitrary",),
        ),
    )(node_feat, p1, p2, q1, q2,
      W_self1, b_self1, W_nbr1, b_nbr1, g1, be1,
      W_self2, b_self2, W_nbr2, b_nbr2, g2, be2)


def kernel(node_feat, neighbor_feats_l1, neighbor_feats_l2,
           W_self1, b_self1, W_nbr1, b_nbr1, g1, be1,
           W_self2, b_self2, W_nbr2, b_nbr2, g2, be2):
    n_nbr = neighbor_feats_l1.shape[0]
    p1, p2 = _sc_reduce(neighbor_feats_l1, neighbor_feats_l2)
    q1, q2 = _tc_reduce(neighbor_feats_l1, neighbor_feats_l2)
    return _tc_dense(node_feat, p1, p2, q1, q2,
                     W_self1, b_self1, W_nbr1, b_nbr1, g1, be1,
                     W_self2, b_self2, W_nbr2, b_nbr2, g2, be2, n_nbr)
